# baseline (speedup 1.0000x reference)
import functools

import jax
import jax.numpy as jnp
from jax import lax
from jax.experimental import pallas as pl
from jax.experimental.pallas import tpu as pltpu
from jax.experimental.pallas import tpu_sc as plsc

_PREC = jax.lax.Precision.DEFAULT
_DN = (((1,), (0,)), ((), ()))


def _dot(a, b):
    return lax.dot_general(a, b, _DN, precision=_PREC,
                           preferred_element_type=jnp.float32)


def _mm_bias(xc, w, bias, relu, Mb):
    B, M, K = xc.shape
    Cout = w.shape[-1]

    def body(x_ref, w_ref, b_ref, o_ref):
        acc = _dot(x_ref[0], w_ref[...]) + b_ref[...]
        if relu:
            acc = jnp.maximum(acc, 0.0)
        o_ref[0] = acc

    return pl.pallas_call(
        body,
        grid=(B, M // Mb),
        in_specs=[
            pl.BlockSpec((1, Mb, K), lambda i, m: (i, m, 0)),
            pl.BlockSpec((K, Cout), lambda i, m: (0, 0)),
            pl.BlockSpec((1, Cout), lambda i, m: (0, 0)),
        ],
        out_specs=pl.BlockSpec((1, Mb, Cout), lambda i, m: (i, m, 0)),
        out_shape=jax.ShapeDtypeStruct((B, M, Cout), jnp.float32),
    )(xc, w, bias)


def _enc2_vq(x2, w2, b2, ct, csq):
    B, Hp = x2.shape[:2]
    Hz = Hp - 1
    K = ct.shape[1]

    def body(xa_ref, xb_ref, w_ref, b_ref, ct_ref, csq_ref, o_ref):
        xa = xa_ref[0, 0]
        xb = xb_ref[0, 0]
        acc0 = _dot(xa, w_ref[0, 0]) + _dot(xb, w_ref[1, 0])
        acc1 = _dot(xa, w_ref[0, 1]) + _dot(xb, w_ref[1, 1])
        z = acc0[0:Hz] + acc1[1:Hz + 1] + b_ref[...]
        s = _dot(z, ct_ref[...])
        dist = jnp.sum(z * z, axis=1, keepdims=True) - 2.0 * s + csq_ref[...]
        minv = jnp.min(dist, axis=1, keepdims=True)
        lane = lax.broadcasted_iota(jnp.int32, dist.shape, 1)
        o_ref[0] = jnp.min(jnp.where(dist == minv, lane, K), axis=1,
                           keepdims=True)

    return pl.pallas_call(
        body,
        grid=(B, Hz),
        in_specs=[
            pl.BlockSpec((1, 1, Hp, 384), lambda i, m: (i, m, 0, 0)),
            pl.BlockSpec((1, 1, Hp, 384), lambda i, m: (i, m + 1, 0, 0)),
            pl.BlockSpec((2, 2, 384, 256), lambda i, m: (0, 0, 0, 0)),
            pl.BlockSpec((1, 256), lambda i, m: (0, 0)),
            pl.BlockSpec((256, K), lambda i, m: (0, 0)),
            pl.BlockSpec((1, K), lambda i, m: (0, 0)),
        ],
        out_specs=pl.BlockSpec((1, Hz, 1), lambda i, m: (i, m, 0)),
        out_shape=jax.ShapeDtypeStruct((B, Hz * Hz, 1), jnp.int32),
    )(x2, x2, w2, b2, ct, csq)


def _sc_gather(table, idx):
    info = plsc.get_sparse_core_info()
    NC, NS = info.num_cores, info.num_subcores
    NW = NC * NS
    CH, CB = 7, 112
    N, D = NW * CH * CB, table.shape[1]
    mesh = plsc.VectorSubcoreMesh(core_axis_name="c", subcore_axis_name="s")

    @functools.partial(
        pl.kernel, mesh=mesh,
        out_type=jax.ShapeDtypeStruct((N, D), jnp.float32),
        scratch_types=[
            pltpu.VMEM((1, CH, CB), jnp.int32),
            pltpu.VMEM((CB, D), jnp.float32),
            pltpu.VMEM((CB, D), jnp.float32),
            pltpu.SemaphoreType.DMA,
            pltpu.SemaphoreType.DMA,
        ],
    )
    def k(table_hbm, idx_hbm, out_hbm, idx_v, rows_a, rows_b, sem_a, sem_b):
        wid = lax.axis_index("s") * NC + lax.axis_index("c")
        base = wid * CH
        pltpu.sync_copy(idx_hbm.at[pl.ds(wid, 1)], idx_v)
        bufs = ((rows_a, sem_a), (rows_b, sem_b))
        cps = [None, None]
        cps[0] = pltpu.async_copy(table_hbm.at[idx_v.at[0, 0]], rows_a, sem_a)
        for c in range(CH):
            if c + 1 < CH:
                rows_n, sem_n = bufs[(c + 1) % 2]
                cps[(c + 1) % 2] = pltpu.async_copy(
                    table_hbm.at[idx_v.at[0, c + 1]], rows_n, sem_n)
            rows, _ = bufs[c % 2]
            cps[c % 2].wait()
            pltpu.sync_copy(rows, out_hbm.at[pl.ds((base + c) * CB, CB)])

    return k(table, idx)


def _dec1(qp, wt, bias):
    B, Hp = qp.shape[:2]
    Hq = Hp - 2
    Cout = wt.shape[-1]

    def body(x0_ref, x1_ref, x2_ref, w_ref, b_ref, o_ref):
        rows = (x0_ref[0, 0], x1_ref[0, 0], x2_ref[0, 0])
        for rh in (0, 1):
            accs = []
            for rw in (0, 1):
                acc = jnp.zeros((Hq, Cout), jnp.float32)
                for b in (0, 1):
                    t = jnp.zeros((Hp, Cout), jnp.float32)
                    for a in (0, 1):
                        t = t + _dot(rows[rh + a], w_ref[2 * a + rh,
                                                         2 * b + rw])
                    acc = acc + t[rw + b:rw + b + Hq]
                accs.append(jnp.maximum(acc + b_ref[...], 0.0))
            pair = jnp.stack(accs, axis=1).reshape(2 * Hq, Cout)
            o_ref[0, rh] = pair

    return pl.pallas_call(
        body,
        grid=(B, Hq),
        in_specs=[
            pl.BlockSpec((1, 1, Hp, 256), lambda i, m: (i, m, 0, 0)),
            pl.BlockSpec((1, 1, Hp, 256), lambda i, m: (i, m + 1, 0, 0)),
            pl.BlockSpec((1, 1, Hp, 256), lambda i, m: (i, m + 2, 0, 0)),
            pl.BlockSpec((4, 4, 256, Cout), lambda i, m: (0, 0, 0, 0)),
            pl.BlockSpec((1, Cout), lambda i, m: (0, 0)),
        ],
        out_specs=pl.BlockSpec((1, 2, 2 * Hq, Cout),
                               lambda i, m: (i, m, 0, 0)),
        out_shape=jax.ShapeDtypeStruct((B, 2 * Hq, 2 * Hq, Cout),
                                       jnp.float32),
    )(qp, qp, qp, wt, bias)


def _dec2(yp, w9, bias):
    B, Hp = yp.shape[:2]
    Hy = Hp - 2

    def body(x0_ref, x1_ref, x2_ref, w_ref, b_ref, o_ref):
        rows = (x0_ref[0, 0], x1_ref[0, 0], x2_ref[0, 0])
        acc = jnp.zeros((Hy, 12), jnp.float32)
        for oh in (0, 1, 2):
            for ow in (0, 1, 2):
                t = _dot(rows[oh], w_ref[oh, ow])
                acc = acc + t[ow:ow + Hy]
        acc = acc + b_ref[...]
        for rh in (0, 1):
            pair = jnp.stack(
                [acc[:, (rh * 2) * 3:(rh * 2) * 3 + 3],
                 acc[:, (rh * 2 + 1) * 3:(rh * 2 + 1) * 3 + 3]],
                axis=1).reshape(2 * Hy, 3)
            o_ref[0, rh] = pair

    return pl.pallas_call(
        body,
        grid=(B, Hy),
        in_specs=[
            pl.BlockSpec((1, 1, Hp, 96), lambda i, m: (i, m, 0, 0)),
            pl.BlockSpec((1, 1, Hp, 96), lambda i, m: (i, m + 1, 0, 0)),
            pl.BlockSpec((1, 1, Hp, 96), lambda i, m: (i, m + 2, 0, 0)),
            pl.BlockSpec((3, 3, 96, 12), lambda i, m: (0, 0, 0, 0)),
            pl.BlockSpec((1, 12), lambda i, m: (0, 0)),
        ],
        out_specs=pl.BlockSpec((1, 2, 2 * Hy, 3), lambda i, m: (i, m, 0, 0)),
        out_shape=jax.ShapeDtypeStruct((B, 2 * Hy, 2 * Hy, 3), jnp.float32),
    )(yp, yp, yp, w9, bias)


def _s2d(x):
    B, Hp, Wp, C = x.shape
    x = x.reshape(B, Hp // 2, 2, Wp // 2, 2, C)
    return x.transpose(0, 1, 3, 2, 4, 5).reshape(B, Hp // 2, Wp // 2, 4 * C)


def _pad1(x):
    return jnp.pad(x, ((0, 0), (1, 1), (1, 1), (0, 0)))


def _enc_w(w):
    O, C = w.shape[:2]
    w = w.reshape(O, C, 2, 2, 2, 2)
    return w.transpose(2, 4, 3, 5, 1, 0).reshape(2, 2, 4 * C, O)


def _dec2_w(w):
    Cout, Cin = w.shape[:2]
    w9 = jnp.zeros((3, 3, Cin, 4 * Cout), jnp.float32)
    for oh in range(3):
        for ow in range(3):
            for rh in range(2):
                for rw in range(2):
                    if (oh - rh) in (0, 1) and (ow - rw) in (0, 1):
                        col = (rh * 2 + rw) * Cout
                        w9 = w9.at[oh, ow, :, col:col + Cout].set(
                            w[:, :, 2 * oh - rh, 2 * ow - rw].T)
    return w9


def kernel(x, enc_w1, enc_b1, enc_w2, enc_b2, codebook, dec_w1, dec_b1,
           dec_w2, dec_b2):
    B = x.shape[0]
    xlp = _pad1(x.transpose(0, 2, 3, 1))
    cols = [xlp[:, kh:kh + 224:2, kw:kw + 224:2, :]
            for kh in range(4) for kw in range(4)]
    xc = jnp.concatenate(cols, axis=-1).reshape(B, 112 * 112, 48)
    w1 = enc_w1.transpose(2, 3, 1, 0).reshape(48, -1)
    y1 = _mm_bias(xc, w1, enc_b1[None, :], True, 1568)
    x2 = _s2d(_pad1(y1.reshape(B, 112, 112, -1)))
    ct = codebook.T
    csq = jnp.sum(codebook * codebook, axis=-1)[None, :]
    idx = _enc2_vq(x2, _enc_w(enc_w2), enc_b2[None, :], ct, csq)
    q = _sc_gather(codebook, idx.reshape(32, 7, 112))
    q = lax.optimization_barrier(q)
    D = codebook.shape[1]
    qz = q.reshape(B, 56, 56, D)
    quantized = qz.transpose(0, 3, 1, 2)
    yd = _dec1(_pad1(qz), dec_w1.transpose(2, 3, 1, 0), dec_b1[None, :])
    dec = _dec2(_pad1(yd), _dec2_w(dec_w2), jnp.tile(dec_b2, 4)[None, :])
    decoded = dec.transpose(0, 3, 1, 2)
    return decoded, quantized


# speedup vs baseline: 1.0746x; 1.0746x over previous
import functools

import jax
import jax.numpy as jnp
from jax import lax
from jax.experimental import pallas as pl
from jax.experimental.pallas import tpu as pltpu
from jax.experimental.pallas import tpu_sc as plsc

_PREC = jax.lax.Precision.DEFAULT
_DN = (((1,), (0,)), ((), ()))


def _dot(a, b):
    return lax.dot_general(a, b, _DN, precision=_PREC,
                           preferred_element_type=jnp.float32)


def _mm_bias(xc, w, bias, relu, Mb):
    B, M, K = xc.shape
    Cout = w.shape[-1]

    def body(x_ref, w_ref, b_ref, o_ref):
        acc = _dot(x_ref[0], w_ref[...]) + b_ref[...]
        if relu:
            acc = jnp.maximum(acc, 0.0)
        o_ref[0] = acc

    return pl.pallas_call(
        body,
        grid=(B, M // Mb),
        in_specs=[
            pl.BlockSpec((1, Mb, K), lambda i, m: (i, m, 0)),
            pl.BlockSpec((K, Cout), lambda i, m: (0, 0)),
            pl.BlockSpec((1, Cout), lambda i, m: (0, 0)),
        ],
        out_specs=pl.BlockSpec((1, Mb, Cout), lambda i, m: (i, m, 0)),
        out_shape=jax.ShapeDtypeStruct((B, M, Cout), jnp.float32),
    )(xc, w, bias)


def _enc2_vq(xab, w2, b2, ct, csq, Mb):
    B, M, Cin = xab[0].shape
    K = ct.shape[1]

    def body(x00, x01, x10, x11, w_ref, b_ref, ct_ref, csq_ref, o_ref):
        z = (_dot(x00[0], w_ref[0, 0]) + _dot(x01[0], w_ref[0, 1])
             + _dot(x10[0], w_ref[1, 0]) + _dot(x11[0], w_ref[1, 1])
             + b_ref[...])
        s = _dot(z, ct_ref[...])
        dist = jnp.sum(z * z, axis=1, keepdims=True) - 2.0 * s + csq_ref[...]
        minv = jnp.min(dist, axis=1, keepdims=True)
        lane = lax.broadcasted_iota(jnp.int32, dist.shape, 1)
        o_ref[0] = jnp.min(jnp.where(dist == minv, lane, K), axis=1,
                           keepdims=True)

    blk = lambda: pl.BlockSpec((1, Mb, Cin), lambda i, m: (i, m, 0))
    return pl.pallas_call(
        body,
        grid=(B, M // Mb),
        in_specs=[
            blk(), blk(), blk(), blk(),
            pl.BlockSpec((2, 2, Cin, 256), lambda i, m: (0, 0, 0, 0)),
            pl.BlockSpec((1, 256), lambda i, m: (0, 0)),
            pl.BlockSpec((256, K), lambda i, m: (0, 0)),
            pl.BlockSpec((1, K), lambda i, m: (0, 0)),
        ],
        out_specs=pl.BlockSpec((1, Mb, 1), lambda i, m: (i, m, 0)),
        out_shape=jax.ShapeDtypeStruct((B, M, 1), jnp.int32),
    )(*xab, w2, b2, ct, csq)


def _sc_gather(table, idx):
    info = plsc.get_sparse_core_info()
    NC, NS = info.num_cores, info.num_subcores
    NW = NC * NS
    CH, CB = 7, 112
    N, D = NW * CH * CB, table.shape[1]
    mesh = plsc.VectorSubcoreMesh(core_axis_name="c", subcore_axis_name="s")

    @functools.partial(
        pl.kernel, mesh=mesh,
        out_type=jax.ShapeDtypeStruct((N, D), jnp.float32),
        scratch_types=[
            pltpu.VMEM((1, CH, CB), jnp.int32),
            pltpu.VMEM((CB, D), jnp.float32),
            pltpu.VMEM((CB, D), jnp.float32),
            pltpu.SemaphoreType.DMA,
            pltpu.SemaphoreType.DMA,
        ],
    )
    def k(table_hbm, idx_hbm, out_hbm, idx_v, rows_a, rows_b, sem_a, sem_b):
        wid = lax.axis_index("s") * NC + lax.axis_index("c")
        base = wid * CH
        pltpu.sync_copy(idx_hbm.at[pl.ds(wid, 1)], idx_v)
        bufs = ((rows_a, sem_a), (rows_b, sem_b))
        cps = [None, None]
        cps[0] = pltpu.async_copy(table_hbm.at[idx_v.at[0, 0]], rows_a, sem_a)
        for c in range(CH):
            if c + 1 < CH:
                rows_n, sem_n = bufs[(c + 1) % 2]
                cps[(c + 1) % 2] = pltpu.async_copy(
                    table_hbm.at[idx_v.at[0, c + 1]], rows_n, sem_n)
            rows, _ = bufs[c % 2]
            cps[c % 2].wait()
            pltpu.sync_copy(rows, out_hbm.at[pl.ds((base + c) * CB, CB)])

    return k(table, idx)


def _dec_conv(yo, w9, bias, Mb, relu):
    B, M, Cin = yo[0].shape
    Cout = w9.shape[-1]

    def body(*refs):
        o_ref, b_ref, w_ref = refs[-1], refs[-2], refs[-3]
        acc = jnp.zeros((Mb, Cout), jnp.float32)
        for oh in (0, 1, 2):
            for ow in (0, 1, 2):
                acc = acc + _dot(refs[oh * 3 + ow][0], w_ref[oh, ow])
        acc = acc + b_ref[...]
        if relu:
            acc = jnp.maximum(acc, 0.0)
        o_ref[0] = acc

    blk = lambda: pl.BlockSpec((1, Mb, Cin), lambda i, m: (i, m, 0))
    return pl.pallas_call(
        body,
        grid=(B, M // Mb),
        in_specs=(
            [blk() for _ in range(9)]
            + [
                pl.BlockSpec((3, 3, Cin, Cout), lambda i, m: (0, 0, 0, 0)),
                pl.BlockSpec((1, Cout), lambda i, m: (0, 0)),
            ]
        ),
        out_specs=pl.BlockSpec((1, Mb, Cout), lambda i, m: (i, m, 0)),
        out_shape=jax.ShapeDtypeStruct((B, M, Cout), jnp.float32),
    )(*yo, w9, bias)


def _s2d(x):
    B, Hp, Wp, C = x.shape
    x = x.reshape(B, Hp // 2, 2, Wp // 2, 2, C)
    return x.transpose(0, 1, 3, 2, 4, 5).reshape(B, Hp // 2, Wp // 2, 4 * C)


def _pad1(x):
    return jnp.pad(x, ((0, 0), (1, 1), (1, 1), (0, 0)))


def _enc_w(w):
    O, C = w.shape[:2]
    w = w.reshape(O, C, 2, 2, 2, 2)
    return w.transpose(2, 4, 3, 5, 1, 0).reshape(2, 2, 4 * C, O)


def _dec2_w(w):
    Cout, Cin = w.shape[:2]
    w9 = jnp.zeros((3, 3, Cin, 4 * Cout), jnp.float32)
    for oh in range(3):
        for ow in range(3):
            for rh in range(2):
                for rw in range(2):
                    if (oh - rh) in (0, 1) and (ow - rw) in (0, 1):
                        col = (rh * 2 + rw) * Cout
                        w9 = w9.at[oh, ow, :, col:col + Cout].set(
                            w[:, :, 2 * oh - rh, 2 * ow - rw].T)
    return w9


def _shift9(xp, Ho):
    B, _, _, C = xp.shape
    return [xp[:, oh:oh + Ho, ow:ow + Ho, :].reshape(B, Ho * Ho, C)
            for oh in range(3) for ow in range(3)]


def kernel(x, enc_w1, enc_b1, enc_w2, enc_b2, codebook, dec_w1, dec_b1,
           dec_w2, dec_b2):
    B = x.shape[0]
    xlp = _pad1(x.transpose(0, 2, 3, 1))
    cols = [xlp[:, kh:kh + 224:2, kw:kw + 224:2, :]
            for kh in range(4) for kw in range(4)]
    xc = jnp.concatenate(cols, axis=-1).reshape(B, 112 * 112, 48)
    w1 = enc_w1.transpose(2, 3, 1, 0).reshape(48, -1)
    y1 = _mm_bias(xc, w1, enc_b1[None, :], True, 1568)
    x2 = _s2d(_pad1(y1.reshape(B, 112, 112, -1)))
    ct = codebook.T
    csq = jnp.sum(codebook * codebook, axis=-1)[None, :]
    xab = [x2[:, a:a + 56, b:b + 56, :].reshape(B, 3136, 384)
           for a in range(2) for b in range(2)]
    idx = _enc2_vq(xab, _enc_w(enc_w2), enc_b2[None, :], ct, csq, 784)
    q = _sc_gather(codebook, idx.reshape(32, 7, 112))
    q = lax.optimization_barrier(q)
    D = codebook.shape[1]
    qz = q.reshape(B, 56, 56, D)
    quantized = qz.transpose(0, 3, 1, 2)
    ph = _dec_conv(_shift9(_pad1(qz), 56), _dec2_w(dec_w1),
                   jnp.tile(dec_b1, 4)[None, :], 784, True)
    yd = (ph.reshape(B, 56, 56, 2, 2, 96)
          .transpose(0, 1, 3, 2, 4, 5).reshape(B, 112, 112, 96))
    p2 = _dec_conv(_shift9(_pad1(yd), 112), _dec2_w(dec_w2),
                   jnp.tile(dec_b2, 4)[None, :], 1568, False)
    decoded = (p2.reshape(B, 112, 112, 2, 2, 3)
               .transpose(0, 5, 1, 3, 2, 4).reshape(B, 3, 224, 224))
    return decoded, quantized


# speedup vs baseline: 1.1602x; 1.0796x over previous
import functools

import jax
import jax.numpy as jnp
from jax import lax
from jax.experimental import pallas as pl
from jax.experimental.pallas import tpu as pltpu
from jax.experimental.pallas import tpu_sc as plsc

_PREC = jax.lax.Precision.DEFAULT
_DN = (((1,), (0,)), ((), ()))


def _dot(a, b):
    return lax.dot_general(a, b, _DN, precision=_PREC,
                           preferred_element_type=jnp.float32)


def _mm_bias(xc, w, bias, relu, Mb):
    B, M, K = xc.shape
    Cout = w.shape[-1]

    def body(x_ref, w_ref, b_ref, o_ref):
        acc = _dot(x_ref[0], w_ref[...]) + b_ref[...]
        if relu:
            acc = jnp.maximum(acc, 0.0)
        o_ref[0] = acc

    return pl.pallas_call(
        body,
        grid=(B, M // Mb),
        in_specs=[
            pl.BlockSpec((1, Mb, K), lambda i, m: (i, m, 0)),
            pl.BlockSpec((K, Cout), lambda i, m: (0, 0)),
            pl.BlockSpec((1, Cout), lambda i, m: (0, 0)),
        ],
        out_specs=pl.BlockSpec((1, Mb, Cout), lambda i, m: (i, m, 0)),
        out_shape=jax.ShapeDtypeStruct((B, M, Cout), jnp.float32),
    )(xc, w, bias)


def _enc2_vq(xab, w2, b2, ct, csq, Mb):
    B, M, Cin = xab[0].shape
    K = ct.shape[1]

    def body(x00, x01, x10, x11, w_ref, b_ref, ct_ref, csq_ref, o_ref):
        z = (_dot(x00[0], w_ref[0, 0]) + _dot(x01[0], w_ref[0, 1])
             + _dot(x10[0], w_ref[1, 0]) + _dot(x11[0], w_ref[1, 1])
             + b_ref[...])
        s = _dot(z, ct_ref[...])
        dist = jnp.sum(z * z, axis=1, keepdims=True) - 2.0 * s + csq_ref[...]
        minv = jnp.min(dist, axis=1, keepdims=True)
        lane = lax.broadcasted_iota(jnp.int32, dist.shape, 1)
        o_ref[0] = jnp.min(jnp.where(dist == minv, lane, K), axis=1,
                           keepdims=True)

    blk = lambda: pl.BlockSpec((1, Mb, Cin), lambda i, m: (i, m, 0))
    return pl.pallas_call(
        body,
        grid=(B, M // Mb),
        in_specs=[
            blk(), blk(), blk(), blk(),
            pl.BlockSpec((2, 2, Cin, 256), lambda i, m: (0, 0, 0, 0)),
            pl.BlockSpec((1, 256), lambda i, m: (0, 0)),
            pl.BlockSpec((256, K), lambda i, m: (0, 0)),
            pl.BlockSpec((1, K), lambda i, m: (0, 0)),
        ],
        out_specs=pl.BlockSpec((1, Mb, 1), lambda i, m: (i, m, 0)),
        out_shape=jax.ShapeDtypeStruct((B, M, 1), jnp.int32),
    )(*xab, w2, b2, ct, csq)


def _sc_gather(table, idx):
    info = plsc.get_sparse_core_info()
    NC, NS = info.num_cores, info.num_subcores
    NW = NC * NS
    CH, CB = 7, 112
    N, D = NW * CH * CB, table.shape[1]
    mesh = plsc.VectorSubcoreMesh(core_axis_name="c", subcore_axis_name="s")

    @functools.partial(
        pl.kernel, mesh=mesh,
        out_type=jax.ShapeDtypeStruct((N, D), jnp.float32),
        scratch_types=[
            pltpu.VMEM((1, CH, CB), jnp.int32),
            pltpu.VMEM((CB, D), jnp.float32),
            pltpu.VMEM((CB, D), jnp.float32),
            pltpu.VMEM((CB, D), jnp.float32),
            pltpu.VMEM((CB, D), jnp.float32),
            pltpu.SemaphoreType.DMA,
            pltpu.SemaphoreType.DMA,
            pltpu.SemaphoreType.DMA,
            pltpu.SemaphoreType.DMA,
        ],
    )
    def k(table_hbm, idx_hbm, out_hbm, idx_v, r0, r1, r2, r3,
          s0, s1, s2, s3):
        wid = lax.axis_index("s") * NC + lax.axis_index("c")
        base = wid * CH
        pltpu.sync_copy(idx_hbm.at[pl.ds(wid, 1)], idx_v)
        NB = 4
        bufs = ((r0, s0), (r1, s1), (r2, s2), (r3, s3))
        cps = [None] * NB
        for c in range(min(NB, CH)):
            rows, sem = bufs[c % NB]
            cps[c % NB] = pltpu.async_copy(
                table_hbm.at[idx_v.at[0, c]], rows, sem)
        for c in range(CH):
            rows, _ = bufs[c % NB]
            cps[c % NB].wait()
            pltpu.sync_copy(rows, out_hbm.at[pl.ds((base + c) * CB, CB)])
            n = c + NB
            if n < CH:
                rows_n, sem_n = bufs[n % NB]
                cps[n % NB] = pltpu.async_copy(
                    table_hbm.at[idx_v.at[0, n]], rows_n, sem_n)

    return k(table, idx)


def _dec_conv(yo, w9, bias, Mb, relu, out_dtype=jnp.float32):
    B, M, Cin = yo[0].shape
    Cout = w9.shape[-1]

    def body(*refs):
        o_ref, b_ref, w_ref = refs[-1], refs[-2], refs[-3]
        acc = jnp.zeros((Mb, Cout), jnp.float32)
        for oh in (0, 1, 2):
            for ow in (0, 1, 2):
                acc = acc + _dot(refs[oh * 3 + ow][0], w_ref[oh, ow])
        acc = acc + b_ref[...]
        if relu:
            acc = jnp.maximum(acc, 0.0)
        o_ref[0] = acc.astype(out_dtype)

    blk = lambda: pl.BlockSpec((1, Mb, Cin), lambda i, m: (i, m, 0))
    return pl.pallas_call(
        body,
        grid=(B, M // Mb),
        in_specs=(
            [blk() for _ in range(9)]
            + [
                pl.BlockSpec((3, 3, Cin, Cout), lambda i, m: (0, 0, 0, 0)),
                pl.BlockSpec((1, Cout), lambda i, m: (0, 0)),
            ]
        ),
        out_specs=pl.BlockSpec((1, Mb, Cout), lambda i, m: (i, m, 0)),
        out_shape=jax.ShapeDtypeStruct((B, M, Cout), out_dtype),
    )(*yo, w9, bias)


def _s2d(x):
    B, Hp, Wp, C = x.shape
    x = x.reshape(B, Hp // 2, 2, Wp // 2, 2, C)
    return x.transpose(0, 1, 3, 2, 4, 5).reshape(B, Hp // 2, Wp // 2, 4 * C)


def _pad1(x):
    return jnp.pad(x, ((0, 0), (1, 1), (1, 1), (0, 0)))


def _enc_w(w):
    O, C = w.shape[:2]
    w = w.reshape(O, C, 2, 2, 2, 2)
    return w.transpose(2, 4, 3, 5, 1, 0).reshape(2, 2, 4 * C, O)


def _dec2_w(w):
    Cout, Cin = w.shape[:2]
    w9 = jnp.zeros((3, 3, Cin, 4 * Cout), jnp.float32)
    for oh in range(3):
        for ow in range(3):
            for rh in range(2):
                for rw in range(2):
                    if (oh - rh) in (0, 1) and (ow - rw) in (0, 1):
                        col = (rh * 2 + rw) * Cout
                        w9 = w9.at[oh, ow, :, col:col + Cout].set(
                            w[:, :, 2 * oh - rh, 2 * ow - rw].T)
    return w9


def _shift9(xp, Ho):
    B, _, _, C = xp.shape
    return [xp[:, oh:oh + Ho, ow:ow + Ho, :].reshape(B, Ho * Ho, C)
            for oh in range(3) for ow in range(3)]


def kernel(x, enc_w1, enc_b1, enc_w2, enc_b2, codebook, dec_w1, dec_b1,
           dec_w2, dec_b2):
    B = x.shape[0]
    xlp = _pad1(x.transpose(0, 2, 3, 1))
    cols = [xlp[:, kh:kh + 224:2, kw:kw + 224:2, :]
            for kh in range(4) for kw in range(4)]
    xc = jnp.concatenate(cols, axis=-1).reshape(B, 112 * 112, 48)
    w1 = enc_w1.transpose(2, 3, 1, 0).reshape(48, -1)
    y1 = _mm_bias(xc, w1, enc_b1[None, :], True, 1568)
    x2 = _s2d(_pad1(y1.reshape(B, 112, 112, -1)))
    ct = codebook.T
    csq = jnp.sum(codebook * codebook, axis=-1)[None, :]
    xab = [x2[:, a:a + 56, b:b + 56, :].reshape(B, 3136, 384)
           for a in range(2) for b in range(2)]
    idx = _enc2_vq(xab, _enc_w(enc_w2), enc_b2[None, :], ct, csq, 784)
    q = _sc_gather(codebook, idx.reshape(32, 7, 112))
    q = lax.optimization_barrier(q)
    D = codebook.shape[1]
    qz = q.reshape(B, 56, 56, D)
    quantized = qz.transpose(0, 3, 1, 2)
    bf = jnp.bfloat16
    qz16 = qz.astype(bf)
    ph = _dec_conv(_shift9(_pad1(qz16), 56), _dec2_w(dec_w1).astype(bf),
                   jnp.tile(dec_b1, 4)[None, :], 784, True, bf)
    yd = (ph.reshape(B, 56, 56, 2, 2, 96)
          .transpose(0, 1, 3, 2, 4, 5).reshape(B, 112, 112, 96))
    p2 = _dec_conv(_shift9(_pad1(yd), 112), _dec2_w(dec_w2).astype(bf),
                   jnp.tile(dec_b2, 4)[None, :], 1568, False)
    decoded = (p2.reshape(B, 112, 112, 2, 2, 3)
               .transpose(0, 5, 1, 3, 2, 4).reshape(B, 3, 224, 224))
    return decoded, quantized


# speedup vs baseline: 1.1636x; 1.0030x over previous
import functools

import jax
import jax.numpy as jnp
from jax import lax
from jax.experimental import pallas as pl
from jax.experimental.pallas import tpu as pltpu
from jax.experimental.pallas import tpu_sc as plsc

_PREC = jax.lax.Precision.DEFAULT
_DN = (((1,), (0,)), ((), ()))


def _dot(a, b):
    return lax.dot_general(a, b, _DN, precision=_PREC,
                           preferred_element_type=jnp.float32)


def _mm_bias(xc, w, bias, relu, Mb):
    B, M, K = xc.shape
    Cout = w.shape[-1]

    def body(x_ref, w_ref, b_ref, o_ref):
        acc = _dot(x_ref[0], w_ref[...]) + b_ref[...]
        if relu:
            acc = jnp.maximum(acc, 0.0)
        o_ref[0] = acc

    return pl.pallas_call(
        body,
        grid=(B, M // Mb),
        in_specs=[
            pl.BlockSpec((1, Mb, K), lambda i, m: (i, m, 0)),
            pl.BlockSpec((K, Cout), lambda i, m: (0, 0)),
            pl.BlockSpec((1, Cout), lambda i, m: (0, 0)),
        ],
        out_specs=pl.BlockSpec((1, Mb, Cout), lambda i, m: (i, m, 0)),
        out_shape=jax.ShapeDtypeStruct((B, M, Cout), jnp.float32),
    )(xc, w, bias)


def _enc2_vq(xab, w2, b2, ct, csq, Mb):
    B, M, Cin = xab[0].shape
    K = ct.shape[1]

    def body(x00, x01, x10, x11, w_ref, b_ref, ct_ref, csq_ref, o_ref):
        z = (_dot(x00[0], w_ref[0, 0]) + _dot(x01[0], w_ref[0, 1])
             + _dot(x10[0], w_ref[1, 0]) + _dot(x11[0], w_ref[1, 1])
             + b_ref[...])
        s = _dot(z, ct_ref[...])
        dist = jnp.sum(z * z, axis=1, keepdims=True) - 2.0 * s + csq_ref[...]
        minv = jnp.min(dist, axis=1, keepdims=True)
        lane = lax.broadcasted_iota(jnp.int32, dist.shape, 1)
        o_ref[0] = jnp.min(jnp.where(dist == minv, lane, K), axis=1,
                           keepdims=True)

    blk = lambda: pl.BlockSpec((1, Mb, Cin), lambda i, m: (i, m, 0))
    return pl.pallas_call(
        body,
        grid=(B, M // Mb),
        in_specs=[
            blk(), blk(), blk(), blk(),
            pl.BlockSpec((2, 2, Cin, 256), lambda i, m: (0, 0, 0, 0)),
            pl.BlockSpec((1, 256), lambda i, m: (0, 0)),
            pl.BlockSpec((256, K), lambda i, m: (0, 0)),
            pl.BlockSpec((1, K), lambda i, m: (0, 0)),
        ],
        out_specs=pl.BlockSpec((1, Mb, 1), lambda i, m: (i, m, 0)),
        out_shape=jax.ShapeDtypeStruct((B, M, 1), jnp.int32),
    )(*xab, w2, b2, ct, csq)


def _sc_gather(table, idx):
    info = plsc.get_sparse_core_info()
    NC, NS = info.num_cores, info.num_subcores
    NW = NC * NS
    CH, CB = 7, 112
    N, D = NW * CH * CB, table.shape[1]
    mesh = plsc.VectorSubcoreMesh(core_axis_name="c", subcore_axis_name="s")

    @functools.partial(
        pl.kernel, mesh=mesh,
        out_type=jax.ShapeDtypeStruct((N, D), jnp.float32),
        scratch_types=[
            pltpu.VMEM((1, CH, CB), jnp.int32),
            pltpu.VMEM((CB, D), jnp.float32),
            pltpu.VMEM((CB, D), jnp.float32),
            pltpu.VMEM((CB, D), jnp.float32),
            pltpu.VMEM((CB, D), jnp.float32),
            pltpu.SemaphoreType.DMA,
            pltpu.SemaphoreType.DMA,
            pltpu.SemaphoreType.DMA,
            pltpu.SemaphoreType.DMA,
        ],
    )
    def k(table_hbm, idx_hbm, out_hbm, idx_v, r0, r1, r2, r3,
          s0, s1, s2, s3):
        wid = lax.axis_index("s") * NC + lax.axis_index("c")
        base = wid * CH
        pltpu.sync_copy(idx_hbm.at[pl.ds(wid, 1)], idx_v)
        NB = 4
        bufs = ((r0, s0), (r1, s1), (r2, s2), (r3, s3))
        cps = [None] * NB
        for c in range(min(NB, CH)):
            rows, sem = bufs[c % NB]
            cps[c % NB] = pltpu.async_copy(
                table_hbm.at[idx_v.at[0, c]], rows, sem)
        for c in range(CH):
            rows, _ = bufs[c % NB]
            cps[c % NB].wait()
            pltpu.sync_copy(rows, out_hbm.at[pl.ds((base + c) * CB, CB)])
            n = c + NB
            if n < CH:
                rows_n, sem_n = bufs[n % NB]
                cps[n % NB] = pltpu.async_copy(
                    table_hbm.at[idx_v.at[0, n]], rows_n, sem_n)

    return k(table, idx)


def _dec_conv(yo, w9, bias, Mb, relu, out_dtype=jnp.float32):
    B, M, Cin = yo[0].shape
    Cout = w9.shape[-1]

    def body(*refs):
        o_ref, b_ref, w_ref = refs[-1], refs[-2], refs[-3]
        acc = jnp.zeros((Mb, Cout), jnp.float32)
        for oh in (0, 1, 2):
            for ow in (0, 1, 2):
                acc = acc + _dot(refs[oh * 3 + ow][0], w_ref[oh, ow])
        acc = acc + b_ref[...]
        if relu:
            acc = jnp.maximum(acc, 0.0)
        o_ref[0] = acc.astype(out_dtype)

    blk = lambda: pl.BlockSpec((1, Mb, Cin), lambda i, m: (i, m, 0))
    return pl.pallas_call(
        body,
        grid=(B, M // Mb),
        in_specs=(
            [blk() for _ in range(9)]
            + [
                pl.BlockSpec((3, 3, Cin, Cout), lambda i, m: (0, 0, 0, 0)),
                pl.BlockSpec((1, Cout), lambda i, m: (0, 0)),
            ]
        ),
        out_specs=pl.BlockSpec((1, Mb, Cout), lambda i, m: (i, m, 0)),
        out_shape=jax.ShapeDtypeStruct((B, M, Cout), out_dtype),
    )(*yo, w9, bias)


def _s2d(x):
    B, Hp, Wp, C = x.shape
    x = x.reshape(B, Hp // 2, 2, Wp // 2, 2, C)
    return x.transpose(0, 1, 3, 2, 4, 5).reshape(B, Hp // 2, Wp // 2, 4 * C)


def _pad1(x):
    return jnp.pad(x, ((0, 0), (1, 1), (1, 1), (0, 0)))


def _enc_w(w):
    O, C = w.shape[:2]
    w = w.reshape(O, C, 2, 2, 2, 2)
    return w.transpose(2, 4, 3, 5, 1, 0).reshape(2, 2, 4 * C, O)


def _dec2_w(w):
    Cout, Cin = w.shape[:2]
    w9 = jnp.zeros((3, 3, Cin, 4 * Cout), jnp.float32)
    for oh in range(3):
        for ow in range(3):
            for rh in range(2):
                for rw in range(2):
                    if (oh - rh) in (0, 1) and (ow - rw) in (0, 1):
                        col = (rh * 2 + rw) * Cout
                        w9 = w9.at[oh, ow, :, col:col + Cout].set(
                            w[:, :, 2 * oh - rh, 2 * ow - rw].T)
    return w9


def _shift9(xp, Ho):
    B, _, _, C = xp.shape
    return [xp[:, oh:oh + Ho, ow:ow + Ho, :].reshape(B, Ho * Ho, C)
            for oh in range(3) for ow in range(3)]


def kernel(x, enc_w1, enc_b1, enc_w2, enc_b2, codebook, dec_w1, dec_b1,
           dec_w2, dec_b2):
    B = x.shape[0]
    xlp = _pad1(x.transpose(0, 2, 3, 1))
    cols = [xlp[:, kh:kh + 224:2, kw:kw + 224:2, :]
            for kh in range(4) for kw in range(4)]
    xc = jnp.concatenate(cols, axis=-1).reshape(B, 112 * 112, 48)
    w1 = enc_w1.transpose(2, 3, 1, 0).reshape(48, -1)
    y1 = _mm_bias(xc, w1, enc_b1[None, :], True, 1568)
    x2 = _s2d(_pad1(y1.reshape(B, 112, 112, -1)))
    ct = codebook.T
    csq = jnp.sum(codebook * codebook, axis=-1)[None, :]
    xab = [x2[:, a:a + 56, b:b + 56, :].reshape(B, 3136, 384)
           for a in range(2) for b in range(2)]
    idx = _enc2_vq(xab, _enc_w(enc_w2), enc_b2[None, :], ct, csq, 1568)
    q = _sc_gather(codebook, idx.reshape(32, 7, 112))
    q = lax.optimization_barrier(q)
    D = codebook.shape[1]
    qz = q.reshape(B, 56, 56, D)
    quantized = qz.transpose(0, 3, 1, 2)
    bf = jnp.bfloat16
    qz16 = qz.astype(bf)
    ph = _dec_conv(_shift9(_pad1(qz16), 56), _dec2_w(dec_w1).astype(bf),
                   jnp.tile(dec_b1, 4)[None, :], 1568, True, bf)
    yd = (ph.reshape(B, 56, 56, 2, 2, 96)
          .transpose(0, 1, 3, 2, 4, 5).reshape(B, 112, 112, 96))
    p2 = _dec_conv(_shift9(_pad1(yd), 112), _dec2_w(dec_w2).astype(bf),
                   jnp.tile(dec_b2, 4)[None, :], 1568, False)
    decoded = (p2.reshape(B, 112, 112, 2, 2, 3)
               .transpose(0, 5, 1, 3, 2, 4).reshape(B, 3, 224, 224))
    return decoded, quantized
